# baseline (speedup 1.0000x reference)
import jax
import jax.numpy as jnp
from jax.experimental import pallas as pl
from jax.experimental.pallas import tpu as pltpu

_NCLS = 1000
_MAXP = 100
_FDIM = 128


def _body(cid_ref, feat_hbm, protos_hbm, counts_hbm, protos_out, counts_out,
          normv, featv, countsv, sem_big, sem_row, sem_f, sem_cin, sem_cout):
    cid = cid_ref[0]

    big = pltpu.make_async_copy(protos_hbm, protos_out, sem_big)
    feat_in = pltpu.make_async_copy(feat_hbm.at[pl.ds(0, 104)], featv, sem_f)
    counts_in = pltpu.make_async_copy(counts_hbm, countsv, sem_cin)
    counts_wr = pltpu.make_async_copy(countsv, counts_out, sem_cout)
    row_wr = pltpu.make_async_copy(normv, protos_out.at[cid], sem_row)

    big.start()
    feat_in.start()
    counts_in.start()

    feat_in.wait()
    f = featv[...]
    norm = jnp.sqrt(jnp.sum(f * f, axis=1, keepdims=True))
    fn = f / jnp.maximum(norm, 1e-12)
    normv[...] = fn[:_MAXP]

    counts_in.wait()
    countsv[pl.ds(cid, 1)] = jnp.ones((1, _MAXP), jnp.int32)
    counts_wr.start()

    big.wait()
    row_wr.start()
    row_wr.wait()
    counts_wr.wait()


def kernel(features, prototypes, counts, class_id):
    cid = jnp.atleast_1d(jnp.asarray(class_id, jnp.int32))
    grid_spec = pltpu.PrefetchScalarGridSpec(
        num_scalar_prefetch=1,
        grid=(1,),
        in_specs=[pl.BlockSpec(memory_space=pltpu.MemorySpace.HBM)] * 3,
        out_specs=[pl.BlockSpec(memory_space=pltpu.MemorySpace.HBM)] * 2,
        scratch_shapes=[
            pltpu.VMEM((_MAXP, _FDIM), jnp.float32),
            pltpu.VMEM((104, _FDIM), jnp.float32),
            pltpu.VMEM((_NCLS, _MAXP), jnp.int32),
            pltpu.SemaphoreType.DMA,
            pltpu.SemaphoreType.DMA,
            pltpu.SemaphoreType.DMA,
            pltpu.SemaphoreType.DMA,
            pltpu.SemaphoreType.DMA,
        ],
    )
    return pl.pallas_call(
        _body,
        grid_spec=grid_spec,
        out_shape=(
            jax.ShapeDtypeStruct((_NCLS, _MAXP, _FDIM), jnp.float32),
            jax.ShapeDtypeStruct((_NCLS, _MAXP), jnp.int32),
        ),
        compiler_params=pltpu.CompilerParams(
            dimension_semantics=("arbitrary",),
        ),
    )(cid, features, prototypes, counts)


# speedup vs baseline: 15.0128x; 15.0128x over previous
import jax
import jax.numpy as jnp
from jax.experimental import pallas as pl
from jax.experimental.pallas import tpu as pltpu

_NCLS = 1000
_MAXP = 100
_FDIM = 128
_CB = 40


def _body(cid_ref, feat_ref, protos_ref, counts_ref, protos_out, counts_out):
    i = pl.program_id(0)
    protos_out[...] = protos_ref[...]
    counts_out[...] = counts_ref[...]
    cid = cid_ref[0]
    base = i * _CB

    @pl.when((cid >= base) & (cid < base + _CB))
    def _():
        f = feat_ref[...]
        norm = jnp.sqrt(jnp.sum(f * f, axis=1, keepdims=True))
        fn = f / jnp.maximum(norm, 1e-12)
        local = cid - base
        protos_out[pl.ds(local, 1)] = fn[:_MAXP][None]
        counts_out[pl.ds(local, 1)] = jnp.ones((1, _MAXP), jnp.int32)


def kernel(features, prototypes, counts, class_id):
    cid = jnp.atleast_1d(jnp.asarray(class_id, jnp.int32))
    grid_spec = pltpu.PrefetchScalarGridSpec(
        num_scalar_prefetch=1,
        grid=(_NCLS // _CB,),
        in_specs=[
            pl.BlockSpec((104, _FDIM), lambda i, s: (0, 0)),
            pl.BlockSpec((_CB, _MAXP, _FDIM), lambda i, s: (i, 0, 0)),
            pl.BlockSpec((_CB, _MAXP), lambda i, s: (i, 0)),
        ],
        out_specs=[
            pl.BlockSpec((_CB, _MAXP, _FDIM), lambda i, s: (i, 0, 0)),
            pl.BlockSpec((_CB, _MAXP), lambda i, s: (i, 0)),
        ],
    )
    return pl.pallas_call(
        _body,
        grid_spec=grid_spec,
        out_shape=(
            jax.ShapeDtypeStruct((_NCLS, _MAXP, _FDIM), jnp.float32),
            jax.ShapeDtypeStruct((_NCLS, _MAXP), jnp.int32),
        ),
        compiler_params=pltpu.CompilerParams(
            dimension_semantics=("arbitrary",),
        ),
    )(cid, features, prototypes, counts)


# speedup vs baseline: 15.7878x; 1.0516x over previous
import jax
import jax.numpy as jnp
from jax.experimental import pallas as pl
from jax.experimental.pallas import tpu as pltpu

_NCLS = 1000
_MAXP = 100
_FDIM = 128
_CC = 25
_K = _NCLS // _CC
_B = 8
_PRE = 6


def _body(cid_ref, feat_hbm, protos_hbm, counts_hbm, protos_out, counts_out,
          featv, normv, countsv, rsems, wsems, sem_f, sem_cin, sem_cout,
          *bufs):
    cid = cid_ref[0]
    c_star = cid // _CC
    local = cid - c_star * _CC

    def rd(k):
        return pltpu.make_async_copy(
            protos_hbm.at[pl.ds(k * _CC, _CC)], bufs[k % _B],
            rsems.at[k % _B])

    def wr(k):
        return pltpu.make_async_copy(
            bufs[k % _B], protos_out.at[pl.ds(k * _CC, _CC)],
            wsems.at[k % _B])

    feat_in = pltpu.make_async_copy(feat_hbm.at[pl.ds(0, 104)], featv, sem_f)
    counts_in = pltpu.make_async_copy(counts_hbm, countsv, sem_cin)
    counts_wr = pltpu.make_async_copy(countsv, counts_out, sem_cout)

    feat_in.start()
    counts_in.start()
    for j in range(_PRE):
        rd(j).start()

    feat_in.wait()
    f = featv[...]
    norm = jnp.sqrt(jnp.sum(f * f, axis=1, keepdims=True))
    normv[...] = (f / jnp.maximum(norm, 1e-12))[:_MAXP]

    for k in range(_K):
        nxt = k + _PRE
        if nxt < _K:
            if nxt >= _B:
                wr(nxt - _B).wait()
            rd(nxt).start()
        rd(k).wait()

        @pl.when(k == c_star)
        def _():
            bufs[k % _B][pl.ds(local, 1)] = normv[...][None]

        wr(k).start()

    counts_in.wait()
    countsv[pl.ds(cid, 1)] = jnp.ones((1, _MAXP), jnp.int32)
    counts_wr.start()

    for k in range(_K - _B, _K):
        wr(k).wait()
    counts_wr.wait()


def kernel(features, prototypes, counts, class_id):
    cid = jnp.atleast_1d(jnp.asarray(class_id, jnp.int32))
    grid_spec = pltpu.PrefetchScalarGridSpec(
        num_scalar_prefetch=1,
        grid=(1,),
        in_specs=[pl.BlockSpec(memory_space=pltpu.MemorySpace.HBM)] * 3,
        out_specs=[pl.BlockSpec(memory_space=pltpu.MemorySpace.HBM)] * 2,
        scratch_shapes=[
            pltpu.VMEM((104, _FDIM), jnp.float32),
            pltpu.VMEM((_MAXP, _FDIM), jnp.float32),
            pltpu.VMEM((_NCLS, _MAXP), jnp.int32),
            pltpu.SemaphoreType.DMA((_B,)),
            pltpu.SemaphoreType.DMA((_B,)),
            pltpu.SemaphoreType.DMA,
            pltpu.SemaphoreType.DMA,
            pltpu.SemaphoreType.DMA,
        ] + [pltpu.VMEM((_CC, _MAXP, _FDIM), jnp.float32)] * _B,
    )
    return pl.pallas_call(
        _body,
        grid_spec=grid_spec,
        out_shape=(
            jax.ShapeDtypeStruct((_NCLS, _MAXP, _FDIM), jnp.float32),
            jax.ShapeDtypeStruct((_NCLS, _MAXP), jnp.int32),
        ),
        compiler_params=pltpu.CompilerParams(
            dimension_semantics=("arbitrary",),
        ),
    )(cid, features, prototypes, counts)


# speedup vs baseline: 16.0770x; 1.0183x over previous
import jax
import jax.numpy as jnp
from jax.experimental import pallas as pl
from jax.experimental.pallas import tpu as pltpu

_NCLS = 1000
_MAXP = 100
_FDIM = 128
_CC = 125
_K = _NCLS // _CC
_B = 4
_PRE = 3


def _body(cid_ref, feat_hbm, protos_hbm, counts_hbm, protos_out, counts_out,
          featv, normv, countsv, rsems, wsems, sem_f, sem_cin, sem_cout,
          *bufs):
    cid = cid_ref[0]
    c_star = cid // _CC
    local = cid - c_star * _CC

    def rd(k):
        return pltpu.make_async_copy(
            protos_hbm.at[pl.ds(k * _CC, _CC)], bufs[k % _B],
            rsems.at[k % _B])

    def wr(k):
        return pltpu.make_async_copy(
            bufs[k % _B], protos_out.at[pl.ds(k * _CC, _CC)],
            wsems.at[k % _B])

    feat_in = pltpu.make_async_copy(feat_hbm.at[pl.ds(0, 104)], featv, sem_f)
    counts_in = pltpu.make_async_copy(counts_hbm, countsv, sem_cin)
    counts_wr = pltpu.make_async_copy(countsv, counts_out, sem_cout)

    feat_in.start()
    counts_in.start()
    for j in range(_PRE):
        rd(j).start()

    feat_in.wait()
    f = featv[...]
    norm = jnp.sqrt(jnp.sum(f * f, axis=1, keepdims=True))
    normv[...] = (f / jnp.maximum(norm, 1e-12))[:_MAXP]

    for k in range(_K):
        nxt = k + _PRE
        if nxt < _K:
            if nxt >= _B:
                wr(nxt - _B).wait()
            rd(nxt).start()
        rd(k).wait()

        @pl.when(k == c_star)
        def _():
            bufs[k % _B][pl.ds(local, 1)] = normv[...][None]

        wr(k).start()

    counts_in.wait()
    countsv[pl.ds(cid, 1)] = jnp.ones((1, _MAXP), jnp.int32)
    counts_wr.start()

    for k in range(_K - _B, _K):
        wr(k).wait()
    counts_wr.wait()


def kernel(features, prototypes, counts, class_id):
    cid = jnp.atleast_1d(jnp.asarray(class_id, jnp.int32))
    grid_spec = pltpu.PrefetchScalarGridSpec(
        num_scalar_prefetch=1,
        grid=(1,),
        in_specs=[pl.BlockSpec(memory_space=pltpu.MemorySpace.HBM)] * 3,
        out_specs=[pl.BlockSpec(memory_space=pltpu.MemorySpace.HBM)] * 2,
        scratch_shapes=[
            pltpu.VMEM((104, _FDIM), jnp.float32),
            pltpu.VMEM((_MAXP, _FDIM), jnp.float32),
            pltpu.VMEM((_NCLS, _MAXP), jnp.int32),
            pltpu.SemaphoreType.DMA((_B,)),
            pltpu.SemaphoreType.DMA((_B,)),
            pltpu.SemaphoreType.DMA,
            pltpu.SemaphoreType.DMA,
            pltpu.SemaphoreType.DMA,
        ] + [pltpu.VMEM((_CC, _MAXP, _FDIM), jnp.float32)] * _B,
    )
    return pl.pallas_call(
        _body,
        grid_spec=grid_spec,
        out_shape=(
            jax.ShapeDtypeStruct((_NCLS, _MAXP, _FDIM), jnp.float32),
            jax.ShapeDtypeStruct((_NCLS, _MAXP), jnp.int32),
        ),
        compiler_params=pltpu.CompilerParams(
            dimension_semantics=("arbitrary",),
        ),
    )(cid, features, prototypes, counts)
